# baseline (speedup 1.0000x reference)
import functools

import jax
import jax.numpy as jnp
from jax import lax
from jax.experimental import pallas as pl
from jax.experimental.pallas import tpu as pltpu
from jax.experimental.pallas import tpu_sc as plsc

NN = 10000
EE = 320000
DD = 128

NC = 2
NS = 16
NW = NC * NS
EW = EE // NW
CH = 80
NCH = EW // CH
RW8 = (NN // NS) // 8 * 8


def _tc1_body(x_ref, w_ref, as_ref, ad_ref, h_ref, aux_ref, gm_ref):
    xb = x_ref[...]
    hb = jnp.dot(xb, w_ref[...], preferred_element_type=jnp.float32)
    h_ref[...] = hb.astype(jnp.bfloat16)
    asr = lax.dot_general(as_ref[...], hb, (((1,), (1,)), ((), ())),
                          preferred_element_type=jnp.float32)
    adr = lax.dot_general(ad_ref[...], hb, (((1,), (1,)), ((), ())),
                          preferred_element_type=jnp.float32)
    aux_ref[...] = jnp.concatenate(
        [asr, adr, jnp.zeros((6, NN), jnp.float32)], axis=0)
    gm_ref[...] = jnp.stack(
        [jnp.full((8, 128), jnp.max(asr), jnp.float32),
         jnp.full((8, 128), jnp.max(adr), jnp.float32)])


def _tc_phase1(x, w, att_s, att_d):
    return pl.pallas_call(
        _tc1_body,
        grid=(1,),
        in_specs=[
            pl.BlockSpec((NN, DD), lambda i: (0, 0)),
            pl.BlockSpec((DD, DD), lambda i: (0, 0)),
            pl.BlockSpec((1, DD), lambda i: (0, 0)),
            pl.BlockSpec((1, DD), lambda i: (0, 0)),
        ],
        out_specs=[
            pl.BlockSpec((NN, DD), lambda i: (0, 0)),
            pl.BlockSpec((8, NN), lambda i: (0, 0)),
            pl.BlockSpec((2, 8, 128), lambda i: (0, 0, 0)),
        ],
        out_shape=[
            jax.ShapeDtypeStruct((NN, DD), jnp.bfloat16),
            jax.ShapeDtypeStruct((8, NN), jnp.float32),
            jax.ShapeDtypeStruct((2, 8, 128), jnp.float32),
        ],
    )(x, w, att_s, att_d)


def _sc_mesh():
    return plsc.VectorSubcoreMesh(core_axis_name="c", subcore_axis_name="s")


@functools.partial(
    pl.kernel,
    mesh=_sc_mesh(),
    compiler_params=pltpu.CompilerParams(needs_layout_passes=False),
    out_type=[
        jax.ShapeDtypeStruct((EE,), jnp.float32),
        jax.ShapeDtypeStruct((NW * NN,), jnp.float32),
    ],
    scratch_types=[
        pltpu.VMEM((NN,), jnp.float32),
        pltpu.VMEM((NN,), jnp.float32),
        pltpu.VMEM((EW,), jnp.int32),
        pltpu.VMEM((EW,), jnp.int32),
        pltpu.VMEM((NN,), jnp.float32),
        pltpu.VMEM((EW,), jnp.float32),
        pltpu.VMEM((16,), jnp.float32),
    ],
)
def _sc_phase2(asf_hbm, adf_hbm, src_hbm, dst_hbm, g_hbm,
               ex_hbm, denp_hbm,
               as_v, ad_v, se_v, de_v, den_v, ex_v, g_v):
    wid = lax.axis_index("s") * NC + lax.axis_index("c")
    base = wid * EW
    pltpu.sync_copy(asf_hbm, as_v)
    pltpu.sync_copy(adf_hbm, ad_v)
    pltpu.sync_copy(src_hbm.at[pl.ds(base, EW)], se_v)
    pltpu.sync_copy(dst_hbm.at[pl.ds(base, EW)], de_v)
    pltpu.sync_copy(g_hbm, g_v)
    gv = g_v[...]
    zero = jnp.zeros((16,), jnp.float32)

    def zbody(i, _):
        den_v[pl.ds(i * 16, 16)] = zero
        return 0

    lax.fori_loop(0, NN // 16, zbody, 0)

    @plsc.parallel_loop(0, EW // 16, unroll=4)
    def _(i):
        sl = pl.ds(i * 16, 16)
        sv = se_v[sl]
        dv = de_v[sl]
        a = plsc.load_gather(as_v, [sv]) + plsc.load_gather(ad_v, [dv])
        a = jnp.where(a >= 0.0, a, a * jnp.float32(0.2))
        e = jnp.exp(a - gv)
        ex_v[sl] = e
        plsc.addupdate_scatter(den_v, [dv], e)
    pltpu.sync_copy(ex_v, ex_hbm.at[pl.ds(base, EW)])
    pltpu.sync_copy(den_v, denp_hbm.at[pl.ds(wid * NN, NN)])


def _tc3_body(dp_ref, dinv_ref):
    s = jnp.sum(dp_ref[...], axis=0, keepdims=True)
    dinv_ref[...] = jnp.broadcast_to(1.0 / (s + 1e-16), dinv_ref.shape)


def _tc_phase3(denp):
    return pl.pallas_call(
        _tc3_body,
        grid=(1,),
        in_specs=[pl.BlockSpec((NW, NN), lambda i: (0, 0))],
        out_specs=pl.BlockSpec((8, NN), lambda i: (0, 0)),
        out_shape=jax.ShapeDtypeStruct((8, NN), jnp.float32),
    )(denp)


@functools.partial(
    pl.kernel,
    mesh=_sc_mesh(),
    compiler_params=pltpu.CompilerParams(needs_layout_passes=False,
                                         use_tc_tiling_on_sc=False),
    out_type=[
        jax.ShapeDtypeStruct((EE,), jnp.float32),
        jax.ShapeDtypeStruct((NC, NN, DD), jnp.float32),
    ],
    scratch_types=[
        pltpu.VMEM((NN,), jnp.float32),
        pltpu.VMEM((CH,), jnp.int32),
        pltpu.VMEM((CH,), jnp.int32),
        pltpu.VMEM((CH,), jnp.int32),
        pltpu.VMEM((CH,), jnp.int32),
        pltpu.VMEM((CH,), jnp.int32),
        pltpu.VMEM((CH,), jnp.int32),
        pltpu.VMEM((CH,), jnp.float32),
        pltpu.VMEM((CH,), jnp.float32),
        pltpu.VMEM((CH,), jnp.float32),
        pltpu.VMEM((CH,), jnp.float32),
        pltpu.VMEM((CH,), jnp.float32),
        pltpu.VMEM((CH,), jnp.float32),
        pltpu.VMEM((CH, DD // 2), jnp.int32),
        pltpu.VMEM((CH, DD // 2), jnp.int32),
        pltpu.VMEM((CH, DD), jnp.float32),
        pltpu.VMEM((CH, DD), jnp.float32),
        pltpu.VMEM_SHARED((NN, DD), jnp.float32),
        pltpu.SemaphoreType.DMA,
        pltpu.SemaphoreType.DMA,
        pltpu.SemaphoreType.DMA,
        pltpu.SemaphoreType.DMA,
        pltpu.SemaphoreType.DMA,
        pltpu.SemaphoreType.DMA,
        pltpu.SemaphoreType.DMA,
        pltpu.SemaphoreType.DMA,
        pltpu.SemaphoreType.DMA,
        pltpu.SemaphoreType.DMA,
    ],
)
def _sc_phase4(h_hbm, dinv_hbm, src_hbm, dst_hbm, ex_hbm,
               al_hbm, outp_hbm,
               dinv_v, sidx0, sidx1, sidx2, didx0, didx1, didx2,
               exch0, exch1, exch2, alch0, alch1, alch2,
               rowsp0, rowsp1, scl0, scl1, acc_sh,
               semi0, semi1, semi2, semg0, semg1,
               sema0, sema1, sema2, semsc0, semsc1):
    cid = lax.axis_index("c")
    tid = lax.axis_index("s")
    wid = tid * NC + cid
    base = wid * EW
    pltpu.sync_copy(dinv_hbm.at[0], dinv_v)
    rowsp_bufs = (rowsp0, rowsp1)
    scl_bufs = (scl0, scl1)
    sidx_bufs = (sidx0, sidx1, sidx2)
    didx_bufs = (didx0, didx1, didx2)
    exch_bufs = (exch0, exch1, exch2)
    alch_bufs = (alch0, alch1, alch2)
    semis = (semi0, semi1, semi2)
    semgs = (semg0, semg1)
    semas = (sema0, sema1, sema2)
    semscs = (semsc0, semsc1)

    zero = jnp.zeros((16,), jnp.float32)

    def zrow(r, _):
        for k in range(DD // 16):
            scl0[r, pl.ds(k * 16, 16)] = zero
        return 0

    lax.fori_loop(0, CH, zrow, 0)
    rstart = tid * RW8
    for q in range(RW8 // CH):
        pltpu.sync_copy(scl0, acc_sh.at[pl.ds(rstart + q * CH, CH)])
    rem = RW8 - (RW8 // CH) * CH
    pltpu.sync_copy(scl0.at[pl.ds(0, rem)],
                    acc_sh.at[pl.ds(rstart + (RW8 // CH) * CH, rem)])

    @pl.when(tid == NS - 1)
    def _():
        pltpu.sync_copy(scl0.at[pl.ds(0, NN - NS * RW8)],
                        acc_sh.at[pl.ds(NS * RW8, NN - NS * RW8)])

    plsc.subcore_barrier()

    def issue_idx(c, b):
        sl = pl.ds(base + c * CH, CH)
        pltpu.async_copy(src_hbm.at[sl], sidx_bufs[b], semis[b])
        pltpu.async_copy(dst_hbm.at[sl], didx_bufs[b], semis[b])
        pltpu.async_copy(ex_hbm.at[sl], exch_bufs[b], semis[b])

    def wait_idx(c, b):
        sl = pl.ds(base + c * CH, CH)
        pltpu.make_async_copy(src_hbm.at[sl], sidx_bufs[b], semis[b]).wait()
        pltpu.make_async_copy(dst_hbm.at[sl], didx_bufs[b], semis[b]).wait()
        pltpu.make_async_copy(ex_hbm.at[sl], exch_bufs[b], semis[b]).wait()

    def issue_gather(sb, gb):
        pltpu.async_copy(h_hbm.at[sidx_bufs[sb]], rowsp_bufs[gb], semgs[gb])

    def wait_gather(sb, gb):
        pltpu.make_async_copy(h_hbm.at[sidx_bufs[sb]], rowsp_bufs[gb],
                              semgs[gb]).wait()

    def wait_scatter(sb, pb):
        pltpu.make_async_copy(scl_bufs[pb], acc_sh.at[didx_bufs[sb]],
                              semscs[pb]).wait()

    def wait_alpha(c, b):
        pltpu.make_async_copy(alch_bufs[b],
                              al_hbm.at[pl.ds(base + c * CH, CH)],
                              semas[b]).wait()

    def step(c, sb, gb, pb):
        c = jnp.asarray(c, jnp.int32)
        sb1 = (sb + 1) % 3
        sb2 = (sb + 2) % 3

        @pl.when(c >= 1)
        def _():
            wait_scatter(sb2, 1 - pb)

        @pl.when(c + 2 <= NCH - 1)
        def _():
            issue_idx(c + 2, sb2)

        @pl.when(c + 1 <= NCH - 1)
        def _():
            wait_idx(c + 1, sb1)
            issue_gather(sb1, 1 - gb)

        wait_gather(sb, gb)

        @pl.when(c >= 3)
        def _():
            wait_alpha(c - 3, sb)

        rowsp_v = rowsp_bufs[gb]
        scl_v = scl_bufs[pb]
        didx_v = didx_bufs[sb]
        exch_v = exch_bufs[sb]
        alch_v = alch_bufs[sb]
        for k in range(CH // 16):
            k16 = pl.ds(k * 16, 16)
            alch_v[k16] = exch_v[k16] * plsc.load_gather(dinv_v,
                                                         [didx_v[k16]])
        pltpu.async_copy(alch_v, al_hbm.at[pl.ds(base + c * CH, CH)],
                         semas[sb])

        @plsc.parallel_loop(0, CH, unroll=2)
        def _(r):
            asp = plsc.load_gather(alch_v, [jnp.full((16,), r, jnp.int32)])
            for k in range(DD // 32):
                w = rowsp_v[r, pl.ds(k * 16, 16)]
                hw = plsc.bitcast(w, jnp.bfloat16)
                lo, hi = plsc.unpack(hw, format=plsc.PackFormat.INTERLEAVED)
                scl_v[r, pl.ds(k * 32, 16)] = lo * asp
                scl_v[r, pl.ds(k * 32 + 16, 16)] = hi * asp

        pltpu.async_copy(scl_v, acc_sh.at[didx_v], semscs[pb], add=True)

    issue_idx(0, 0)
    issue_idx(1, 1)
    wait_idx(0, 0)
    issue_gather(0, 0)

    def chunk6(c6, _):
        for u in range(6):
            step(c6 * 6 + u, u % 3, u % 2, u % 2)
        return 0

    lax.fori_loop(0, NCH // 6, chunk6, 0)
    for cc in range((NCH // 6) * 6, NCH):
        step(cc, cc % 3, cc % 2, cc % 2)
    wait_scatter((NCH - 1) % 3, (NCH - 1) % 2)
    for cc in range(NCH - 3, NCH):
        wait_alpha(cc, cc % 3)
    plsc.subcore_barrier()
    pltpu.sync_copy(acc_sh.at[pl.ds(rstart, RW8)],
                    outp_hbm.at[cid, pl.ds(rstart, RW8)])

    @pl.when(tid == NS - 1)
    def _():
        pltpu.sync_copy(acc_sh.at[pl.ds(NS * RW8, NN - NS * RW8)],
                        outp_hbm.at[cid, pl.ds(NS * RW8, NN - NS * RW8)])


def _tc5_body(p_ref, perm_ref, b_ref, o_ref):
    s = p_ref[0] + p_ref[1]
    o_ref[...] = jnp.dot(s, perm_ref[...],
                         preferred_element_type=jnp.float32) + b_ref[...]


def _tc_phase5(outp, perm, bias2):
    blk = 1000
    grid = NN // blk
    return pl.pallas_call(
        _tc5_body,
        grid=(grid,),
        in_specs=[
            pl.BlockSpec((NC, blk, DD), lambda i: (0, i, 0)),
            pl.BlockSpec((DD, DD), lambda i: (0, 0)),
            pl.BlockSpec((1, DD), lambda i: (0, 0)),
        ],
        out_specs=pl.BlockSpec((blk, DD), lambda i: (i, 0)),
        out_shape=jax.ShapeDtypeStruct((NN, DD), jnp.float32),
    )(outp, perm, bias2)


def _perm_matrix():
    import numpy as np
    p = np.zeros((DD, DD), dtype=np.float32)
    for k in range(DD // 32):
        for j in range(16):
            p[32 * k + j, 32 * k + 2 * j] = 1.0
            p[32 * k + 16 + j, 32 * k + 2 * j + 1] = 1.0
    return p


_PERM = _perm_matrix()


def kernel(x, edge_index, W, att_src, att_dst, bias):
    att_s = att_src.reshape(1, DD)
    att_d = att_dst.reshape(1, DD)
    src = edge_index[0]
    dst = edge_index[1]

    h, aux, gm = _tc_phase1(x, W, att_s, att_d)
    g = gm[0, 0, 0] + gm[1, 0, 0]
    g = jnp.where(g >= 0.0, g, g * jnp.float32(0.2))
    gvec = jnp.full((16,), g, jnp.float32)

    ex, denp = _sc_phase2(aux[0], aux[1], src, dst, gvec)
    dinv = _tc_phase3(denp.reshape(NW, NN))

    hp = jax.lax.bitcast_convert_type(h.reshape(NN, DD // 2, 2), jnp.int32)
    alpha, outp = _sc_phase4(hp, dinv, src, dst, ex)

    out = _tc_phase5(outp, _PERM, bias.reshape(1, DD))
    return out, edge_index, alpha.reshape(EE, 1)


# speedup vs baseline: 1.0270x; 1.0270x over previous
import functools

import jax
import jax.numpy as jnp
from jax import lax
from jax.experimental import pallas as pl
from jax.experimental.pallas import tpu as pltpu
from jax.experimental.pallas import tpu_sc as plsc

NN = 10000
EE = 320000
DD = 128

NC = 2
NS = 16
NW = NC * NS
EW = EE // NW
CH = 80
NCH = EW // CH
RW8 = (NN // NS) // 8 * 8


def _tc1_body(x_ref, w_ref, as_ref, ad_ref, h_ref, aux_ref, gm_ref):
    xb = x_ref[...]
    hb = jnp.dot(xb, w_ref[...], preferred_element_type=jnp.float32)
    h_ref[...] = hb
    asr = lax.dot_general(as_ref[...], hb, (((1,), (1,)), ((), ())),
                          preferred_element_type=jnp.float32)
    adr = lax.dot_general(ad_ref[...], hb, (((1,), (1,)), ((), ())),
                          preferred_element_type=jnp.float32)
    aux_ref[...] = jnp.concatenate(
        [asr, adr, jnp.zeros((6, NN), jnp.float32)], axis=0)
    gm_ref[...] = jnp.stack(
        [jnp.full((8, 128), jnp.max(asr), jnp.float32),
         jnp.full((8, 128), jnp.max(adr), jnp.float32)])


def _tc_phase1(x, w, att_s, att_d):
    return pl.pallas_call(
        _tc1_body,
        grid=(1,),
        in_specs=[
            pl.BlockSpec((NN, DD), lambda i: (0, 0)),
            pl.BlockSpec((DD, DD), lambda i: (0, 0)),
            pl.BlockSpec((1, DD), lambda i: (0, 0)),
            pl.BlockSpec((1, DD), lambda i: (0, 0)),
        ],
        out_specs=[
            pl.BlockSpec((NN, DD), lambda i: (0, 0)),
            pl.BlockSpec((8, NN), lambda i: (0, 0)),
            pl.BlockSpec((2, 8, 128), lambda i: (0, 0, 0)),
        ],
        out_shape=[
            jax.ShapeDtypeStruct((NN, DD), jnp.float32),
            jax.ShapeDtypeStruct((8, NN), jnp.float32),
            jax.ShapeDtypeStruct((2, 8, 128), jnp.float32),
        ],
    )(x, w, att_s, att_d)


def _sc_mesh():
    return plsc.VectorSubcoreMesh(core_axis_name="c", subcore_axis_name="s")


@functools.partial(
    pl.kernel,
    mesh=_sc_mesh(),
    compiler_params=pltpu.CompilerParams(needs_layout_passes=False),
    out_type=[
        jax.ShapeDtypeStruct((EE,), jnp.float32),
        jax.ShapeDtypeStruct((NW * NN,), jnp.float32),
    ],
    scratch_types=[
        pltpu.VMEM((NN,), jnp.float32),
        pltpu.VMEM((NN,), jnp.float32),
        pltpu.VMEM((EW,), jnp.int32),
        pltpu.VMEM((EW,), jnp.int32),
        pltpu.VMEM((NN,), jnp.float32),
        pltpu.VMEM((EW,), jnp.float32),
        pltpu.VMEM((16,), jnp.float32),
    ],
)
def _sc_phase2(asf_hbm, adf_hbm, src_hbm, dst_hbm, g_hbm,
               ex_hbm, denp_hbm,
               as_v, ad_v, se_v, de_v, den_v, ex_v, g_v):
    wid = lax.axis_index("s") * NC + lax.axis_index("c")
    base = wid * EW
    pltpu.sync_copy(asf_hbm, as_v)
    pltpu.sync_copy(adf_hbm, ad_v)
    pltpu.sync_copy(src_hbm.at[pl.ds(base, EW)], se_v)
    pltpu.sync_copy(dst_hbm.at[pl.ds(base, EW)], de_v)
    pltpu.sync_copy(g_hbm, g_v)
    gv = g_v[...]
    zero = jnp.zeros((16,), jnp.float32)

    def zbody(i, _):
        den_v[pl.ds(i * 16, 16)] = zero
        return 0

    lax.fori_loop(0, NN // 16, zbody, 0)

    @plsc.parallel_loop(0, EW // 16, unroll=4)
    def _(i):
        sl = pl.ds(i * 16, 16)
        sv = se_v[sl]
        dv = de_v[sl]
        a = plsc.load_gather(as_v, [sv]) + plsc.load_gather(ad_v, [dv])
        a = jnp.where(a >= 0.0, a, a * jnp.float32(0.2))
        e = jnp.exp(a - gv)
        ex_v[sl] = e
        plsc.addupdate_scatter(den_v, [dv], e)
    pltpu.sync_copy(ex_v, ex_hbm.at[pl.ds(base, EW)])
    pltpu.sync_copy(den_v, denp_hbm.at[pl.ds(wid * NN, NN)])


def _tc3_body(dp_ref, dinv_ref):
    s = jnp.sum(dp_ref[...], axis=0, keepdims=True)
    dinv_ref[...] = jnp.broadcast_to(1.0 / (s + 1e-16), dinv_ref.shape)


def _tc_phase3(denp):
    return pl.pallas_call(
        _tc3_body,
        grid=(1,),
        in_specs=[pl.BlockSpec((NW, NN), lambda i: (0, 0))],
        out_specs=pl.BlockSpec((8, NN), lambda i: (0, 0)),
        out_shape=jax.ShapeDtypeStruct((8, NN), jnp.float32),
    )(denp)


@functools.partial(
    pl.kernel,
    mesh=_sc_mesh(),
    compiler_params=pltpu.CompilerParams(needs_layout_passes=False),
    out_type=[
        jax.ShapeDtypeStruct((EE,), jnp.float32),
        jax.ShapeDtypeStruct((NC, NN, DD), jnp.float32),
    ],
    scratch_types=[
        pltpu.VMEM((NN,), jnp.float32),
        pltpu.VMEM((CH,), jnp.int32),
        pltpu.VMEM((CH,), jnp.int32),
        pltpu.VMEM((CH,), jnp.int32),
        pltpu.VMEM((CH,), jnp.int32),
        pltpu.VMEM((CH,), jnp.int32),
        pltpu.VMEM((CH,), jnp.int32),
        pltpu.VMEM((CH,), jnp.float32),
        pltpu.VMEM((CH,), jnp.float32),
        pltpu.VMEM((CH,), jnp.float32),
        pltpu.VMEM((CH,), jnp.float32),
        pltpu.VMEM((CH,), jnp.float32),
        pltpu.VMEM((CH,), jnp.float32),
        pltpu.VMEM((CH, DD), jnp.float32),
        pltpu.VMEM((CH, DD), jnp.float32),
        pltpu.VMEM((CH, DD), jnp.float32),
        pltpu.VMEM_SHARED((NN, DD), jnp.float32),
        pltpu.SemaphoreType.DMA,
        pltpu.SemaphoreType.DMA,
        pltpu.SemaphoreType.DMA,
        pltpu.SemaphoreType.DMA,
        pltpu.SemaphoreType.DMA,
        pltpu.SemaphoreType.DMA,
        pltpu.SemaphoreType.DMA,
        pltpu.SemaphoreType.DMA,
        pltpu.SemaphoreType.DMA,
        pltpu.SemaphoreType.DMA,
        pltpu.SemaphoreType.DMA,
        pltpu.SemaphoreType.DMA,
    ],
)
def _sc_phase4(h_hbm, dinv_hbm, src_hbm, dst_hbm, ex_hbm,
               al_hbm, outp_hbm,
               dinv_v, sidx0, sidx1, sidx2, didx0, didx1, didx2,
               exch0, exch1, exch2, alch0, alch1, alch2,
               rows0_v, rows1_v, rows2_v, acc_sh,
               semi0, semi1, semi2, semg0, semg1, semg2,
               sema0, sema1, sema2, semsc0, semsc1, semsc2):
    cid = lax.axis_index("c")
    tid = lax.axis_index("s")
    wid = tid * NC + cid
    base = wid * EW
    pltpu.sync_copy(dinv_hbm.at[0], dinv_v)
    rows_bufs = (rows0_v, rows1_v, rows2_v)
    sidx_bufs = (sidx0, sidx1, sidx2)
    didx_bufs = (didx0, didx1, didx2)
    exch_bufs = (exch0, exch1, exch2)
    alch_bufs = (alch0, alch1, alch2)
    semis = (semi0, semi1, semi2)
    semgs = (semg0, semg1, semg2)
    semas = (sema0, sema1, sema2)
    semscs = (semsc0, semsc1, semsc2)

    zero = jnp.zeros((16,), jnp.float32)

    def zrow(r, _):
        for k in range(DD // 16):
            rows0_v[r, pl.ds(k * 16, 16)] = zero
        return 0

    lax.fori_loop(0, CH, zrow, 0)
    rstart = tid * RW8
    for q in range(RW8 // CH):
        pltpu.sync_copy(rows0_v, acc_sh.at[pl.ds(rstart + q * CH, CH)])
    rem = RW8 - (RW8 // CH) * CH
    pltpu.sync_copy(rows0_v.at[pl.ds(0, rem)],
                    acc_sh.at[pl.ds(rstart + (RW8 // CH) * CH, rem)])

    @pl.when(tid == NS - 1)
    def _():
        pltpu.sync_copy(rows0_v.at[pl.ds(0, NN - NS * RW8)],
                        acc_sh.at[pl.ds(NS * RW8, NN - NS * RW8)])

    plsc.subcore_barrier()

    def issue_idx(c, b):
        sl = pl.ds(base + c * CH, CH)
        pltpu.async_copy(src_hbm.at[sl], sidx_bufs[b], semis[b])
        pltpu.async_copy(dst_hbm.at[sl], didx_bufs[b], semis[b])
        pltpu.async_copy(ex_hbm.at[sl], exch_bufs[b], semis[b])

    def wait_idx(c, b):
        sl = pl.ds(base + c * CH, CH)
        pltpu.make_async_copy(src_hbm.at[sl], sidx_bufs[b], semis[b]).wait()
        pltpu.make_async_copy(dst_hbm.at[sl], didx_bufs[b], semis[b]).wait()
        pltpu.make_async_copy(ex_hbm.at[sl], exch_bufs[b], semis[b]).wait()

    def issue_gather(b):
        pltpu.async_copy(h_hbm.at[sidx_bufs[b]], rows_bufs[b], semgs[b])

    def wait_gather(b):
        pltpu.make_async_copy(h_hbm.at[sidx_bufs[b]], rows_bufs[b],
                              semgs[b]).wait()

    def wait_scatter(b):
        pltpu.make_async_copy(rows_bufs[b], acc_sh.at[didx_bufs[b]],
                              semscs[b]).wait()

    def wait_alpha(c, b):
        pltpu.make_async_copy(alch_bufs[b],
                              al_hbm.at[pl.ds(base + c * CH, CH)],
                              semas[b]).wait()

    def step(c, b):
        c = jnp.asarray(c, jnp.int32)
        b1 = (b + 1) % 3
        b2 = (b + 2) % 3

        @pl.when(c >= 1)
        def _():
            wait_scatter(b2)

        @pl.when(c + 2 <= NCH - 1)
        def _():
            issue_idx(c + 2, b2)

        @pl.when(c + 1 <= NCH - 1)
        def _():
            wait_idx(c + 1, b1)
            issue_gather(b1)

        @pl.when(c >= 3)
        def _():
            wait_alpha(c - 3, b)

        rows_v = rows_bufs[b]
        didx_v = didx_bufs[b]
        exch_v = exch_bufs[b]
        alch_v = alch_bufs[b]
        for k in range(CH // 16):
            k16 = pl.ds(k * 16, 16)
            alch_v[k16] = exch_v[k16] * plsc.load_gather(dinv_v,
                                                         [didx_v[k16]])
        pltpu.async_copy(alch_v, al_hbm.at[pl.ds(base + c * CH, CH)],
                         semas[b])
        wait_gather(b)

        @plsc.parallel_loop(0, CH, unroll=4)
        def _(r):
            asp = plsc.load_gather(alch_v, [jnp.full((16,), r, jnp.int32)])
            for k in range(DD // 16):
                k16 = pl.ds(k * 16, 16)
                rows_v[r, k16] = rows_v[r, k16] * asp

        pltpu.async_copy(rows_v, acc_sh.at[didx_v], semscs[b], add=True)

    issue_idx(0, 0)
    issue_idx(1, 1)
    wait_idx(0, 0)
    issue_gather(0)

    def chunk3(c3, _):
        for bb in range(3):
            step(c3 * 3 + bb, bb)
        return 0

    lax.fori_loop(0, NCH // 3, chunk3, 0)
    for cc in range((NCH // 3) * 3, NCH):
        step(cc, cc % 3)
    wait_scatter((NCH - 1) % 3)
    for cc in range(NCH - 3, NCH):
        wait_alpha(cc, cc % 3)
    plsc.subcore_barrier()
    pltpu.sync_copy(acc_sh.at[pl.ds(rstart, RW8)],
                    outp_hbm.at[cid, pl.ds(rstart, RW8)])

    @pl.when(tid == NS - 1)
    def _():
        pltpu.sync_copy(acc_sh.at[pl.ds(NS * RW8, NN - NS * RW8)],
                        outp_hbm.at[cid, pl.ds(NS * RW8, NN - NS * RW8)])


def _tc5_body(p_ref, b_ref, o_ref):
    o_ref[...] = p_ref[0] + p_ref[1] + b_ref[...]


def _tc_phase5(outp, bias2):
    blk = 1000
    grid = NN // blk
    return pl.pallas_call(
        _tc5_body,
        grid=(grid,),
        in_specs=[
            pl.BlockSpec((NC, blk, DD), lambda i: (0, i, 0)),
            pl.BlockSpec((1, DD), lambda i: (0, 0)),
        ],
        out_specs=pl.BlockSpec((blk, DD), lambda i: (i, 0)),
        out_shape=jax.ShapeDtypeStruct((NN, DD), jnp.float32),
    )(outp, bias2)


def kernel(x, edge_index, W, att_src, att_dst, bias):
    att_s = att_src.reshape(1, DD)
    att_d = att_dst.reshape(1, DD)
    src = edge_index[0]
    dst = edge_index[1]

    h, aux, gm = _tc_phase1(x, W, att_s, att_d)
    g = gm[0, 0, 0] + gm[1, 0, 0]
    g = jnp.where(g >= 0.0, g, g * jnp.float32(0.2))
    gvec = jnp.full((16,), g, jnp.float32)

    ex, denp = _sc_phase2(aux[0], aux[1], src, dst, gvec)
    dinv = _tc_phase3(denp.reshape(NW, NN))

    alpha, outp = _sc_phase4(h, dinv, src, dst, ex)

    out = _tc_phase5(outp, bias.reshape(1, DD))
    return out, edge_index, alpha.reshape(EE, 1)


# speedup vs baseline: 1.0377x; 1.0104x over previous
import functools

import jax
import jax.numpy as jnp
from jax import lax
from jax.experimental import pallas as pl
from jax.experimental.pallas import tpu as pltpu
from jax.experimental.pallas import tpu_sc as plsc

NN = 10000
EE = 320000
DD = 128

NC = 2
NS = 16
NW = NC * NS
EW = EE // NW
CH = 80
NCH = EW // CH
RW8 = (NN // NS) // 8 * 8


def _tc1_body(x_ref, w_ref, as_ref, ad_ref, h_ref, aux_ref, gm_ref):
    xb = x_ref[...]
    hb = jnp.dot(xb, w_ref[...], preferred_element_type=jnp.float32)
    h_ref[...] = hb
    asr = lax.dot_general(as_ref[...], hb, (((1,), (1,)), ((), ())),
                          preferred_element_type=jnp.float32)
    adr = lax.dot_general(ad_ref[...], hb, (((1,), (1,)), ((), ())),
                          preferred_element_type=jnp.float32)
    aux_ref[...] = jnp.concatenate(
        [asr, adr, jnp.zeros((6, NN), jnp.float32)], axis=0)
    gm_ref[...] = jnp.stack(
        [jnp.full((8, 128), jnp.max(asr), jnp.float32),
         jnp.full((8, 128), jnp.max(adr), jnp.float32)])


def _tc_phase1(x, w, att_s, att_d):
    return pl.pallas_call(
        _tc1_body,
        grid=(1,),
        in_specs=[
            pl.BlockSpec((NN, DD), lambda i: (0, 0)),
            pl.BlockSpec((DD, DD), lambda i: (0, 0)),
            pl.BlockSpec((1, DD), lambda i: (0, 0)),
            pl.BlockSpec((1, DD), lambda i: (0, 0)),
        ],
        out_specs=[
            pl.BlockSpec((NN, DD), lambda i: (0, 0)),
            pl.BlockSpec((8, NN), lambda i: (0, 0)),
            pl.BlockSpec((2, 8, 128), lambda i: (0, 0, 0)),
        ],
        out_shape=[
            jax.ShapeDtypeStruct((NN, DD), jnp.float32),
            jax.ShapeDtypeStruct((8, NN), jnp.float32),
            jax.ShapeDtypeStruct((2, 8, 128), jnp.float32),
        ],
    )(x, w, att_s, att_d)


def _sc_mesh():
    return plsc.VectorSubcoreMesh(core_axis_name="c", subcore_axis_name="s")


@functools.partial(
    pl.kernel,
    mesh=_sc_mesh(),
    compiler_params=pltpu.CompilerParams(needs_layout_passes=False),
    out_type=[
        jax.ShapeDtypeStruct((EE,), jnp.float32),
        jax.ShapeDtypeStruct((NW * NN,), jnp.float32),
    ],
    scratch_types=[
        pltpu.VMEM((NN,), jnp.float32),
        pltpu.VMEM((NN,), jnp.float32),
        pltpu.VMEM((EW,), jnp.int32),
        pltpu.VMEM((EW,), jnp.int32),
        pltpu.VMEM((NN,), jnp.float32),
        pltpu.VMEM((EW,), jnp.float32),
        pltpu.VMEM((16,), jnp.float32),
        pltpu.SemaphoreType.DMA,
    ],
)
def _sc_phase2(asf_hbm, adf_hbm, src_hbm, dst_hbm, g_hbm,
               ex_hbm, denp_hbm,
               as_v, ad_v, se_v, de_v, den_v, ex_v, g_v, sem):
    wid = lax.axis_index("s") * NC + lax.axis_index("c")
    base = wid * EW
    pltpu.async_copy(asf_hbm, as_v, sem)
    pltpu.async_copy(adf_hbm, ad_v, sem)
    pltpu.async_copy(src_hbm.at[pl.ds(base, EW)], se_v, sem)
    pltpu.async_copy(dst_hbm.at[pl.ds(base, EW)], de_v, sem)
    pltpu.async_copy(g_hbm, g_v, sem)
    pltpu.make_async_copy(asf_hbm, as_v, sem).wait()
    pltpu.make_async_copy(adf_hbm, ad_v, sem).wait()
    pltpu.make_async_copy(src_hbm.at[pl.ds(base, EW)], se_v, sem).wait()
    pltpu.make_async_copy(dst_hbm.at[pl.ds(base, EW)], de_v, sem).wait()
    pltpu.make_async_copy(g_hbm, g_v, sem).wait()
    gv = g_v[...]
    zero = jnp.zeros((16,), jnp.float32)

    def zbody(i, _):
        den_v[pl.ds(i * 16, 16)] = zero
        return 0

    lax.fori_loop(0, NN // 16, zbody, 0)

    @plsc.parallel_loop(0, EW // 16, unroll=4)
    def _(i):
        sl = pl.ds(i * 16, 16)
        sv = se_v[sl]
        dv = de_v[sl]
        a = plsc.load_gather(as_v, [sv]) + plsc.load_gather(ad_v, [dv])
        a = jnp.where(a >= 0.0, a, a * jnp.float32(0.2))
        e = jnp.exp(a - gv)
        ex_v[sl] = e
        plsc.addupdate_scatter(den_v, [dv], e)
    pltpu.sync_copy(ex_v, ex_hbm.at[pl.ds(base, EW)])
    pltpu.sync_copy(den_v, denp_hbm.at[pl.ds(wid * NN, NN)])


def _tc3_body(dp_ref, dinv_ref):
    s = jnp.sum(dp_ref[...], axis=0, keepdims=True)
    dinv_ref[...] = jnp.broadcast_to(1.0 / (s + 1e-16), dinv_ref.shape)


def _tc_phase3(denp):
    return pl.pallas_call(
        _tc3_body,
        grid=(1,),
        in_specs=[pl.BlockSpec((NW, NN), lambda i: (0, 0))],
        out_specs=pl.BlockSpec((8, NN), lambda i: (0, 0)),
        out_shape=jax.ShapeDtypeStruct((8, NN), jnp.float32),
    )(denp)


@functools.partial(
    pl.kernel,
    mesh=_sc_mesh(),
    compiler_params=pltpu.CompilerParams(needs_layout_passes=False),
    out_type=[
        jax.ShapeDtypeStruct((EE,), jnp.float32),
        jax.ShapeDtypeStruct((NC, NN, DD), jnp.float32),
    ],
    scratch_types=[
        pltpu.VMEM((NN,), jnp.float32),
        pltpu.VMEM((CH,), jnp.int32),
        pltpu.VMEM((CH,), jnp.int32),
        pltpu.VMEM((CH,), jnp.int32),
        pltpu.VMEM((CH,), jnp.int32),
        pltpu.VMEM((CH,), jnp.int32),
        pltpu.VMEM((CH,), jnp.int32),
        pltpu.VMEM((CH,), jnp.float32),
        pltpu.VMEM((CH,), jnp.float32),
        pltpu.VMEM((CH,), jnp.float32),
        pltpu.VMEM((CH,), jnp.float32),
        pltpu.VMEM((CH,), jnp.float32),
        pltpu.VMEM((CH,), jnp.float32),
        pltpu.VMEM((CH, DD), jnp.float32),
        pltpu.VMEM((CH, DD), jnp.float32),
        pltpu.VMEM((CH, DD), jnp.float32),
        pltpu.VMEM_SHARED((NN, DD), jnp.float32),
        pltpu.SemaphoreType.DMA,
        pltpu.SemaphoreType.DMA,
        pltpu.SemaphoreType.DMA,
        pltpu.SemaphoreType.DMA,
        pltpu.SemaphoreType.DMA,
        pltpu.SemaphoreType.DMA,
        pltpu.SemaphoreType.DMA,
        pltpu.SemaphoreType.DMA,
        pltpu.SemaphoreType.DMA,
        pltpu.SemaphoreType.DMA,
        pltpu.SemaphoreType.DMA,
        pltpu.SemaphoreType.DMA,
    ],
)
def _sc_phase4(h_hbm, dinv_hbm, src_hbm, dst_hbm, ex_hbm,
               al_hbm, outp_hbm,
               dinv_v, sidx0, sidx1, sidx2, didx0, didx1, didx2,
               exch0, exch1, exch2, alch0, alch1, alch2,
               rows0_v, rows1_v, rows2_v, acc_sh,
               semi0, semi1, semi2, semg0, semg1, semg2,
               sema0, sema1, sema2, semsc0, semsc1, semsc2):
    cid = lax.axis_index("c")
    tid = lax.axis_index("s")
    wid = tid * NC + cid
    base = wid * EW
    pltpu.sync_copy(dinv_hbm.at[0], dinv_v)
    rows_bufs = (rows0_v, rows1_v, rows2_v)
    sidx_bufs = (sidx0, sidx1, sidx2)
    didx_bufs = (didx0, didx1, didx2)
    exch_bufs = (exch0, exch1, exch2)
    alch_bufs = (alch0, alch1, alch2)
    semis = (semi0, semi1, semi2)
    semgs = (semg0, semg1, semg2)
    semas = (sema0, sema1, sema2)
    semscs = (semsc0, semsc1, semsc2)

    zero = jnp.zeros((16,), jnp.float32)

    def zrow(r, _):
        for k in range(DD // 16):
            rows0_v[r, pl.ds(k * 16, 16)] = zero
        return 0

    lax.fori_loop(0, CH, zrow, 0)
    rstart = tid * RW8
    for q in range(RW8 // CH):
        pltpu.sync_copy(rows0_v, acc_sh.at[pl.ds(rstart + q * CH, CH)])
    rem = RW8 - (RW8 // CH) * CH
    pltpu.sync_copy(rows0_v.at[pl.ds(0, rem)],
                    acc_sh.at[pl.ds(rstart + (RW8 // CH) * CH, rem)])

    @pl.when(tid == NS - 1)
    def _():
        pltpu.sync_copy(rows0_v.at[pl.ds(0, NN - NS * RW8)],
                        acc_sh.at[pl.ds(NS * RW8, NN - NS * RW8)])

    plsc.subcore_barrier()

    def issue_idx(c, b):
        sl = pl.ds(base + c * CH, CH)
        pltpu.async_copy(src_hbm.at[sl], sidx_bufs[b], semis[b])
        pltpu.async_copy(dst_hbm.at[sl], didx_bufs[b], semis[b])
        pltpu.async_copy(ex_hbm.at[sl], exch_bufs[b], semis[b])

    def wait_idx(c, b):
        sl = pl.ds(base + c * CH, CH)
        pltpu.make_async_copy(src_hbm.at[sl], sidx_bufs[b], semis[b]).wait()
        pltpu.make_async_copy(dst_hbm.at[sl], didx_bufs[b], semis[b]).wait()
        pltpu.make_async_copy(ex_hbm.at[sl], exch_bufs[b], semis[b]).wait()

    def issue_gather(b):
        pltpu.async_copy(h_hbm.at[sidx_bufs[b]], rows_bufs[b], semgs[b])

    def wait_gather(b):
        pltpu.make_async_copy(h_hbm.at[sidx_bufs[b]], rows_bufs[b],
                              semgs[b]).wait()

    def wait_scatter(b):
        pltpu.make_async_copy(rows_bufs[b], acc_sh.at[didx_bufs[b]],
                              semscs[b]).wait()

    def wait_alpha(c, b):
        pltpu.make_async_copy(alch_bufs[b],
                              al_hbm.at[pl.ds(base + c * CH, CH)],
                              semas[b]).wait()

    def step(c, b):
        c = jnp.asarray(c, jnp.int32)
        b1 = (b + 1) % 3
        b2 = (b + 2) % 3

        @pl.when(c >= 1)
        def _():
            wait_scatter(b2)

        @pl.when(c + 2 <= NCH - 1)
        def _():
            issue_idx(c + 2, b2)

        @pl.when(c + 1 <= NCH - 1)
        def _():
            wait_idx(c + 1, b1)
            issue_gather(b1)

        @pl.when(c >= 3)
        def _():
            wait_alpha(c - 3, b)

        rows_v = rows_bufs[b]
        didx_v = didx_bufs[b]
        exch_v = exch_bufs[b]
        alch_v = alch_bufs[b]
        for k in range(CH // 16):
            k16 = pl.ds(k * 16, 16)
            alch_v[k16] = exch_v[k16] * plsc.load_gather(dinv_v,
                                                         [didx_v[k16]])
        pltpu.async_copy(alch_v, al_hbm.at[pl.ds(base + c * CH, CH)],
                         semas[b])
        wait_gather(b)

        @plsc.parallel_loop(0, CH, unroll=4)
        def _(r):
            asp = plsc.load_gather(alch_v, [jnp.full((16,), r, jnp.int32)])
            for k in range(DD // 16):
                k16 = pl.ds(k * 16, 16)
                rows_v[r, k16] = rows_v[r, k16] * asp

        pltpu.async_copy(rows_v, acc_sh.at[didx_v], semscs[b], add=True)

    issue_idx(0, 0)
    issue_idx(1, 1)
    wait_idx(0, 0)
    issue_gather(0)

    def chunk3(c3, _):
        for bb in range(3):
            step(c3 * 3 + bb, bb)
        return 0

    lax.fori_loop(0, NCH // 3, chunk3, 0)
    for cc in range((NCH // 3) * 3, NCH):
        step(cc, cc % 3)
    wait_scatter((NCH - 1) % 3)
    for cc in range(NCH - 3, NCH):
        wait_alpha(cc, cc % 3)
    plsc.subcore_barrier()
    pltpu.sync_copy(acc_sh.at[pl.ds(rstart, RW8)],
                    outp_hbm.at[cid, pl.ds(rstart, RW8)])

    @pl.when(tid == NS - 1)
    def _():
        pltpu.sync_copy(acc_sh.at[pl.ds(NS * RW8, NN - NS * RW8)],
                        outp_hbm.at[cid, pl.ds(NS * RW8, NN - NS * RW8)])


def _tc5_body(p_ref, b_ref, o_ref):
    o_ref[...] = p_ref[0] + p_ref[1] + b_ref[...]


def _tc_phase5(outp, bias2):
    blk = 1000
    grid = NN // blk
    return pl.pallas_call(
        _tc5_body,
        grid=(grid,),
        in_specs=[
            pl.BlockSpec((NC, blk, DD), lambda i: (0, i, 0)),
            pl.BlockSpec((1, DD), lambda i: (0, 0)),
        ],
        out_specs=pl.BlockSpec((blk, DD), lambda i: (i, 0)),
        out_shape=jax.ShapeDtypeStruct((NN, DD), jnp.float32),
    )(outp, bias2)


def kernel(x, edge_index, W, att_src, att_dst, bias):
    att_s = att_src.reshape(1, DD)
    att_d = att_dst.reshape(1, DD)
    src = edge_index[0]
    dst = edge_index[1]

    h, aux, gm = _tc_phase1(x, W, att_s, att_d)
    g = gm[0, 0, 0] + gm[1, 0, 0]
    g = jnp.where(g >= 0.0, g, g * jnp.float32(0.2))
    gvec = jnp.full((16,), g, jnp.float32)

    ex, denp = _sc_phase2(aux[0], aux[1], src, dst, gvec)
    dinv = _tc_phase3(denp.reshape(NW, NN))

    alpha, outp = _sc_phase4(h, dinv, src, dst, ex)

    out = _tc_phase5(outp, bias.reshape(1, DD))
    return out, edge_index, alpha.reshape(EE, 1)
